# Initial kernel scaffold; baseline (speedup 1.0000x reference)
import jax
import jax.numpy as jnp
from jax.experimental import pallas as pl


def kernel(xyz, points, W0, b0, gamma0, beta0, W1, b1, gamma1, beta1, W2, b2, gamma2, beta2):
    raise NotImplementedError("write your pallas kernel here")


# speedup vs baseline: 1.7005x; 1.7005x over previous
import functools

import jax
import jax.numpy as jnp
from jax.experimental import pallas as pl
from jax.experimental.pallas import tpu as pltpu

_NPOINT = 1024
_RADIUS = 0.1
_NSAMPLE = 32
_BN_EPS = 1e-5

_B = 4
_N = 8192
_ROWS = 64
_COLS = 128


def _fps_body(x_ref, y_ref, z_ref, nx_ref, ny_ref, nz_ref):
    iota = (jax.lax.broadcasted_iota(jnp.int32, (_ROWS, _COLS), 0) * _COLS
            + jax.lax.broadcasted_iota(jnp.int32, (_ROWS, _COLS), 1))

    xs = [x_ref[b] for b in range(_B)]
    ys = [y_ref[b] for b in range(_B)]
    zs = [z_ref[b] for b in range(_B)]

    def body(i, state):
        dists, farthest = state
        new_d = []
        new_f = []
        for b in range(_B):
            oh = iota == farthest[b]
            cx = jnp.sum(jnp.where(oh, xs[b], 0.0))
            cy = jnp.sum(jnp.where(oh, ys[b], 0.0))
            cz = jnp.sum(jnp.where(oh, zs[b], 0.0))
            nx_ref[b, i] = cx
            ny_ref[b, i] = cy
            nz_ref[b, i] = cz
            dx = xs[b] - cx
            dy = ys[b] - cy
            dz = zs[b] - cz
            d = dx * dx + dy * dy
            d = d + dz * dz
            db = jnp.minimum(dists[b], d)
            m = jnp.max(db)
            idx = jnp.min(jnp.where(db == m, iota, _N))
            new_d.append(db)
            new_f.append(idx)
        return new_d, new_f

    dists0 = [jnp.full((_ROWS, _COLS), 1e10, dtype=jnp.float32) for _ in range(_B)]
    far0 = [jnp.int32(0) for _ in range(_B)]
    jax.lax.fori_loop(0, _NPOINT, body, (dists0, far0))


def _fps_new_xyz(xyz):
    xt = xyz.transpose(0, 2, 1).reshape(_B, 3, _ROWS, _COLS)
    x, y, z = xt[:, 0], xt[:, 1], xt[:, 2]
    out_sds = jax.ShapeDtypeStruct((_B, _NPOINT), jnp.float32)
    smem_spec = pl.BlockSpec(memory_space=pltpu.SMEM)
    nx, ny, nz = pl.pallas_call(
        _fps_body,
        out_shape=(out_sds, out_sds, out_sds),
        out_specs=(smem_spec, smem_spec, smem_spec),
    )(x, y, z)
    return jnp.stack([nx, ny, nz], axis=-1)


def _ball_query(xyz, new_xyz):
    B, N, _ = xyz.shape
    qq = jnp.sum(new_xyz ** 2, axis=-1)[:, :, None]
    pp = jnp.sum(xyz ** 2, axis=-1)[:, None, :]
    d2 = qq + pp - 2.0 * jnp.einsum('bsd,bnd->bsn', new_xyz, xyz)
    in_ball = d2 < _RADIUS ** 2
    arangeN = jnp.arange(N, dtype=jnp.int32)[None, None, :]
    key = jnp.where(in_ball, arangeN, N)
    neg_topk, _ = jax.lax.top_k(-key, _NSAMPLE)
    cand = -neg_topk
    first = cand[..., :1]
    first = jnp.where(first >= N, 0, first)
    idx = jnp.where(cand < N, cand, first)
    return idx.astype(jnp.int32)


def kernel(xyz, points, W0, b0, gamma0, beta0, W1, b1, gamma1, beta1, W2, b2, gamma2, beta2):
    new_xyz = _fps_new_xyz(xyz)
    idx = _ball_query(xyz, new_xyz)
    B = idx.shape[0]
    bidx = jnp.arange(B)[:, None, None]
    g_xyz = xyz[bidx, idx] - new_xyz[:, :, None, :]
    g_pts = points[bidx, idx]
    x = jnp.concatenate([g_xyz, g_pts], axis=-1)
    for (W, b, g, be) in ((W0, b0, gamma0, beta0), (W1, b1, gamma1, beta1),
                          (W2, b2, gamma2, beta2)):
        x = x @ W + b
        mean = jnp.mean(x, axis=(0, 1, 2))
        var = jnp.var(x, axis=(0, 1, 2))
        x = (x - mean) / jnp.sqrt(var + _BN_EPS) * g + be
        x = jax.nn.relu(x)
    new_points = jnp.max(x, axis=2)
    return new_xyz, new_points


# speedup vs baseline: 3.3685x; 1.9808x over previous
import functools

import jax
import jax.numpy as jnp
from jax.experimental import pallas as pl
from jax.experimental.pallas import tpu as pltpu

_NPOINT = 1024
_RADIUS = 0.1
_NSAMPLE = 32
_BN_EPS = 1e-5

_B = 4
_N = 8192
_ROWS = 64
_COLS = 128


def _fps_body(x_ref, y_ref, z_ref, nx_ref, ny_ref, nz_ref):
    iota = (jax.lax.broadcasted_iota(jnp.int32, (_ROWS, _COLS), 0) * _COLS
            + jax.lax.broadcasted_iota(jnp.int32, (_ROWS, _COLS), 1))

    xs = [x_ref[b] for b in range(_B)]
    ys = [y_ref[b] for b in range(_B)]
    zs = [z_ref[b] for b in range(_B)]

    def body(i, state):
        dists, farthest = state
        new_d = []
        new_f = []
        for b in range(_B):
            oh = iota == farthest[b]
            cx = jnp.sum(jnp.where(oh, xs[b], 0.0))
            cy = jnp.sum(jnp.where(oh, ys[b], 0.0))
            cz = jnp.sum(jnp.where(oh, zs[b], 0.0))
            nx_ref[b, i] = cx
            ny_ref[b, i] = cy
            nz_ref[b, i] = cz
            dx = xs[b] - cx
            dy = ys[b] - cy
            dz = zs[b] - cz
            d = dx * dx + dy * dy
            d = d + dz * dz
            db = jnp.minimum(dists[b], d)
            m = jnp.max(db)
            idx = jnp.min(jnp.where(db == m, iota, _N))
            new_d.append(db)
            new_f.append(idx)
        return new_d, new_f

    dists0 = [jnp.full((_ROWS, _COLS), 1e10, dtype=jnp.float32) for _ in range(_B)]
    far0 = [jnp.int32(0) for _ in range(_B)]
    jax.lax.fori_loop(0, _NPOINT, body, (dists0, far0))


def _fps_new_xyz(xyz):
    xt = xyz.transpose(0, 2, 1).reshape(_B, 3, _ROWS, _COLS)
    x, y, z = xt[:, 0], xt[:, 1], xt[:, 2]
    out_sds = jax.ShapeDtypeStruct((_B, _NPOINT), jnp.float32)
    smem_spec = pl.BlockSpec(memory_space=pltpu.SMEM)
    nx, ny, nz = pl.pallas_call(
        _fps_body,
        out_shape=(out_sds, out_sds, out_sds),
        out_specs=(smem_spec, smem_spec, smem_spec),
    )(x, y, z)
    return jnp.stack([nx, ny, nz], axis=-1)


_QT = 128


def _bq_body(q_ref, p_ref, out_ref):
    q = q_ref[0]
    qx, qy, qz = q[:, 0:1], q[:, 1:2], q[:, 2:3]
    px = p_ref[0, 0:1, :]
    py = p_ref[0, 1:2, :]
    pz = p_ref[0, 2:3, :]
    qq = (qx * qx + qy * qy) + qz * qz
    pp = (px * px + py * py) + pz * pz
    qp = jax.lax.dot_general(q, p_ref[0], (((1,), (0,)), ((), ())),
                             preferred_element_type=jnp.float32)
    d2 = (qq + pp) - 2.0 * qp
    mask = jnp.where(d2 < _RADIUS * _RADIUS, 1.0, 0.0)

    ch = 128
    nch = _N // ch
    li = jax.lax.broadcasted_iota(jnp.int32, (ch, ch), 0)
    lj = jax.lax.broadcasted_iota(jnp.int32, (ch, ch), 1)
    ltri = jnp.where(li <= lj, 1.0, 0.0)
    base = jnp.zeros((_QT, 1), jnp.float32)
    psums = []
    for c in range(nch):
        mc = mask[:, c * ch:(c + 1) * ch]
        lsum = jax.lax.dot(mc, ltri, precision=jax.lax.Precision.HIGHEST)
        psums.append(lsum + base)
        base = base + lsum[:, ch - 1:ch]
    psum = jnp.concatenate(psums, axis=-1)
    cnt = base

    u = jnp.minimum(psum, 33.0)
    cols = [jnp.sum(jnp.where(u <= float(k), 1.0, 0.0), axis=-1)
            for k in range(_NSAMPLE)]
    idx = jnp.stack(cols, axis=-1)
    first = jnp.where(cnt > 0.0, idx[:, 0:1], 0.0)
    krange = jax.lax.broadcasted_iota(jnp.int32, (_QT, _NSAMPLE), 1)
    out_ref[0] = jnp.where(krange < cnt.astype(jnp.int32), idx, first).astype(jnp.int32)


def _ball_query(xyz, new_xyz):
    xt = xyz.transpose(0, 2, 1)
    grid = (_B, _NPOINT // _QT)
    return pl.pallas_call(
        _bq_body,
        grid=grid,
        in_specs=[
            pl.BlockSpec((1, _QT, 3), lambda b, s: (b, s, 0)),
            pl.BlockSpec((1, 3, _N), lambda b, s: (b, 0, 0)),
        ],
        out_specs=pl.BlockSpec((1, _QT, _NSAMPLE), lambda b, s: (b, s, 0)),
        out_shape=jax.ShapeDtypeStruct((_B, _NPOINT, _NSAMPLE), jnp.int32),
    )(new_xyz, xt)


def kernel(xyz, points, W0, b0, gamma0, beta0, W1, b1, gamma1, beta1, W2, b2, gamma2, beta2):
    new_xyz = _fps_new_xyz(xyz)
    idx = _ball_query(xyz, new_xyz)
    B = idx.shape[0]
    bidx = jnp.arange(B)[:, None, None]
    g_xyz = xyz[bidx, idx] - new_xyz[:, :, None, :]
    g_pts = points[bidx, idx]
    x = jnp.concatenate([g_xyz, g_pts], axis=-1)
    for (W, b, g, be) in ((W0, b0, gamma0, beta0), (W1, b1, gamma1, beta1),
                          (W2, b2, gamma2, beta2)):
        x = x @ W + b
        mean = jnp.mean(x, axis=(0, 1, 2))
        var = jnp.var(x, axis=(0, 1, 2))
        x = (x - mean) / jnp.sqrt(var + _BN_EPS) * g + be
        x = jax.nn.relu(x)
    new_points = jnp.max(x, axis=2)
    return new_xyz, new_points


# speedup vs baseline: 8.9868x; 2.6679x over previous
import functools

import jax
import jax.numpy as jnp
from jax.experimental import pallas as pl
from jax.experimental.pallas import tpu as pltpu

_NPOINT = 1024
_RADIUS = 0.1
_NSAMPLE = 32
_BN_EPS = 1e-5

_B = 4
_N = 8192
_ROWS = 64
_COLS = 128


def _fps_body(x_ref, y_ref, z_ref, nx_ref, ny_ref, nz_ref):
    iota = (jax.lax.broadcasted_iota(jnp.int32, (_ROWS, _COLS), 0) * _COLS
            + jax.lax.broadcasted_iota(jnp.int32, (_ROWS, _COLS), 1))

    xs = [x_ref[b] for b in range(_B)]
    ys = [y_ref[b] for b in range(_B)]
    zs = [z_ref[b] for b in range(_B)]

    def body(i, state):
        dists, farthest = state
        new_d = []
        new_f = []
        for b in range(_B):
            oh = iota == farthest[b]
            cx = jnp.sum(jnp.where(oh, xs[b], 0.0))
            cy = jnp.sum(jnp.where(oh, ys[b], 0.0))
            cz = jnp.sum(jnp.where(oh, zs[b], 0.0))
            nx_ref[b, i] = cx
            ny_ref[b, i] = cy
            nz_ref[b, i] = cz
            dx = xs[b] - cx
            dy = ys[b] - cy
            dz = zs[b] - cz
            d = dx * dx + dy * dy
            d = d + dz * dz
            db = jnp.minimum(dists[b], d)
            m = jnp.max(db)
            idx = jnp.min(jnp.where(db == m, iota, _N))
            new_d.append(db)
            new_f.append(idx)
        return new_d, new_f

    dists0 = [jnp.full((_ROWS, _COLS), 1e10, dtype=jnp.float32) for _ in range(_B)]
    far0 = [jnp.int32(0) for _ in range(_B)]
    jax.lax.fori_loop(0, _NPOINT, body, (dists0, far0))


def _fps_new_xyz(xyz):
    xt = xyz.transpose(0, 2, 1).reshape(_B, 3, _ROWS, _COLS)
    x, y, z = xt[:, 0], xt[:, 1], xt[:, 2]
    out_sds = jax.ShapeDtypeStruct((_B, _NPOINT), jnp.float32)
    smem_spec = pl.BlockSpec(memory_space=pltpu.SMEM)
    nx, ny, nz = pl.pallas_call(
        _fps_body,
        out_shape=(out_sds, out_sds, out_sds),
        out_specs=(smem_spec, smem_spec, smem_spec),
    )(x, y, z)
    return jnp.stack([nx, ny, nz], axis=-1)


_QT = 128


def _bq_body(q_ref, p_ref, out_ref):
    q = q_ref[0]
    qx, qy, qz = q[:, 0:1], q[:, 1:2], q[:, 2:3]
    px = p_ref[0, 0:1, :]
    py = p_ref[0, 1:2, :]
    pz = p_ref[0, 2:3, :]
    qq = (qx * qx + qy * qy) + qz * qz
    pp = (px * px + py * py) + pz * pz
    qp = jax.lax.dot_general(q, p_ref[0], (((1,), (0,)), ((), ())),
                             preferred_element_type=jnp.float32)
    d2 = (qq + pp) - 2.0 * qp
    mask = jnp.where(d2 < _RADIUS * _RADIUS, 1.0, 0.0)

    ch = 128
    nch = _N // ch
    li = jax.lax.broadcasted_iota(jnp.int32, (ch, ch), 0)
    lj = jax.lax.broadcasted_iota(jnp.int32, (ch, ch), 1)
    ltri = jnp.where(li <= lj, 1.0, 0.0)
    base = jnp.zeros((_QT, 1), jnp.float32)
    psums = []
    for c in range(nch):
        mc = mask[:, c * ch:(c + 1) * ch]
        lsum = jax.lax.dot(mc, ltri, precision=jax.lax.Precision.HIGHEST)
        psums.append(lsum + base)
        base = base + lsum[:, ch - 1:ch]
    psum = jnp.concatenate(psums, axis=-1)
    cnt = base

    u = jnp.minimum(psum, 33.0)
    cols = [jnp.sum(jnp.where(u <= float(k), 1.0, 0.0), axis=-1)
            for k in range(_NSAMPLE)]
    idx = jnp.stack(cols, axis=-1)
    first = jnp.where(cnt > 0.0, idx[:, 0:1], 0.0)
    krange = jax.lax.broadcasted_iota(jnp.int32, (_QT, _NSAMPLE), 1)
    out_ref[0] = jnp.where(krange < cnt.astype(jnp.int32), idx, first).astype(jnp.int32)


def _ball_query(xyz, new_xyz):
    xt = xyz.transpose(0, 2, 1)
    grid = (_B, _NPOINT // _QT)
    return pl.pallas_call(
        _bq_body,
        grid=grid,
        in_specs=[
            pl.BlockSpec((1, _QT, 3), lambda b, s: (b, s, 0)),
            pl.BlockSpec((1, 3, _N), lambda b, s: (b, 0, 0)),
        ],
        out_specs=pl.BlockSpec((1, _QT, _NSAMPLE), lambda b, s: (b, s, 0)),
        out_shape=jax.ShapeDtypeStruct((_B, _NPOINT, _NSAMPLE), jnp.int32),
    )(new_xyz, xt)


def _prep_body(cat_ref, q_ref, w0_ref, b0_ref, p_out, q_out):
    p_out[0] = jax.lax.dot_general(
        cat_ref[0], w0_ref[...], (((1,), (0,)), ((), ())),
        preferred_element_type=jnp.float32) + b0_ref[...]
    q_out[0] = jax.lax.dot_general(
        q_ref[0], w0_ref[0:3, :], (((1,), (0,)), ((), ())),
        preferred_element_type=jnp.float32)


def _prep(xyz, points, new_xyz, W0, b0):
    cat = jnp.concatenate([xyz, points], axis=-1)
    return pl.pallas_call(
        _prep_body,
        grid=(_B,),
        in_specs=[
            pl.BlockSpec((1, _N, 19), lambda b: (b, 0, 0)),
            pl.BlockSpec((1, _NPOINT, 3), lambda b: (b, 0, 0)),
            pl.BlockSpec((19, 32), lambda b: (0, 0)),
            pl.BlockSpec((32,), lambda b: (0,)),
        ],
        out_specs=(
            pl.BlockSpec((1, _N, 32), lambda b: (b, 0, 0)),
            pl.BlockSpec((1, _NPOINT, 32), lambda b: (b, 0, 0)),
        ),
        out_shape=(
            jax.ShapeDtypeStruct((_B, _N, 32), jnp.float32),
            jax.ShapeDtypeStruct((_B, _NPOINT, 32), jnp.float32),
        ),
    )(cat, new_xyz, W0, b0)


_RT = 2048
_R = _B * _NPOINT * _NSAMPLE


def _bn_stats(s_ref, cin):
    n = float(_R)
    mean = s_ref[0:1, :] / n
    var = s_ref[1:2, :] / n - mean * mean
    inv = jax.lax.rsqrt(var + _BN_EPS)
    return mean, inv


def _mlp_mid_body(z_ref, s_ref, w_ref, b_ref, g_ref, be_ref, out_ref,
                  sout_ref, acc_ref):
    t = pl.program_id(0)

    @pl.when(t == 0)
    def _():
        acc_ref[...] = jnp.zeros_like(acc_ref)

    z = z_ref[...]
    mean, inv = _bn_stats(s_ref, z.shape[-1])
    a = jnp.maximum((z - mean) * inv * g_ref[...] + be_ref[...], 0.0)
    z2 = jax.lax.dot_general(a, w_ref[...], (((1,), (0,)), ((), ())),
                             preferred_element_type=jnp.float32) + b_ref[...]
    out_ref[...] = z2
    acc_ref[0:1, :] += jnp.sum(z2, axis=0, keepdims=True)
    acc_ref[1:2, :] += jnp.sum(z2 * z2, axis=0, keepdims=True)

    @pl.when(t == pl.num_programs(0) - 1)
    def _():
        sout_ref[...] = acc_ref[...]


def _mlp_mid(z1, stats1, W1, b1, g1, be1):
    cout = W1.shape[1]
    grid = (_R // _RT,)
    return pl.pallas_call(
        _mlp_mid_body,
        grid=grid,
        in_specs=[
            pl.BlockSpec((_RT, 32), lambda t: (t, 0)),
            pl.BlockSpec((2, 32), lambda t: (0, 0)),
            pl.BlockSpec((32, cout), lambda t: (0, 0)),
            pl.BlockSpec((cout,), lambda t: (0,)),
            pl.BlockSpec((cout,), lambda t: (0,)),
            pl.BlockSpec((cout,), lambda t: (0,)),
        ],
        out_specs=(
            pl.BlockSpec((_RT, cout), lambda t: (t, 0)),
            pl.BlockSpec((2, cout), lambda t: (0, 0)),
        ),
        out_shape=(
            jax.ShapeDtypeStruct((_R, cout), jnp.float32),
            jax.ShapeDtypeStruct((2, cout), jnp.float32),
        ),
        scratch_shapes=[pltpu.VMEM((2, cout), jnp.float32)],
    )(z1, stats1, W1, b1, g1, be1)


_QT3 = 64


def _mlp_last_body(z_ref, s_ref, w_ref, b_ref, g_ref, be_ref, out_ref,
                   sout_ref, acc_ref):
    t = pl.program_id(0)

    @pl.when(t == 0)
    def _():
        acc_ref[...] = jnp.zeros_like(acc_ref)

    z = z_ref[...].reshape(_QT3 * _NSAMPLE, 32)
    mean, inv = _bn_stats(s_ref, 32)
    a = jnp.maximum((z - mean) * inv * g_ref[...] + be_ref[...], 0.0)
    z3 = jax.lax.dot_general(a, w_ref[...], (((1,), (0,)), ((), ())),
                             preferred_element_type=jnp.float32) + b_ref[...]
    acc_ref[0:1, :] += jnp.sum(z3, axis=0, keepdims=True)
    acc_ref[1:2, :] += jnp.sum(z3 * z3, axis=0, keepdims=True)
    out_ref[...] = jnp.max(z3.reshape(_QT3, _NSAMPLE, 64), axis=1)

    @pl.when(t == pl.num_programs(0) - 1)
    def _():
        sout_ref[...] = acc_ref[...]


def _mlp_last(z2, stats2, W2, b2, g2, be2):
    grid = (_B * _NPOINT // _QT3,)
    z2r = z2.reshape(_B * _NPOINT, _NSAMPLE, 32)
    return pl.pallas_call(
        _mlp_last_body,
        grid=grid,
        in_specs=[
            pl.BlockSpec((_QT3, _NSAMPLE, 32), lambda t: (t, 0, 0)),
            pl.BlockSpec((2, 32), lambda t: (0, 0)),
            pl.BlockSpec((32, 64), lambda t: (0, 0)),
            pl.BlockSpec((64,), lambda t: (0,)),
            pl.BlockSpec((32,), lambda t: (0,)),
            pl.BlockSpec((32,), lambda t: (0,)),
        ],
        out_specs=(
            pl.BlockSpec((_QT3, 64), lambda t: (t, 0)),
            pl.BlockSpec((2, 64), lambda t: (0, 0)),
        ),
        out_shape=(
            jax.ShapeDtypeStruct((_B * _NPOINT, 64), jnp.float32),
            jax.ShapeDtypeStruct((2, 64), jnp.float32),
        ),
        scratch_shapes=[pltpu.VMEM((2, 64), jnp.float32)],
    )(z2r, stats2, W2, b2, g2, be2)


def _final_body(m_ref, s_ref, g_ref, be_ref, out_ref):
    mean, inv = _bn_stats(s_ref, 64)
    out_ref[...] = jnp.maximum(
        (m_ref[...] - mean) * inv * g_ref[...] + be_ref[...], 0.0)


def _final(m, stats3, g2, be2):
    return pl.pallas_call(
        _final_body,
        in_specs=[
            pl.BlockSpec((_B * _NPOINT, 64), lambda: (0, 0)),
            pl.BlockSpec((2, 64), lambda: (0, 0)),
            pl.BlockSpec((64,), lambda: (0,)),
            pl.BlockSpec((64,), lambda: (0,)),
        ],
        out_specs=pl.BlockSpec((_B * _NPOINT, 64), lambda: (0, 0)),
        out_shape=jax.ShapeDtypeStruct((_B * _NPOINT, 64), jnp.float32),
    )(m, stats3, g2, be2)


def kernel(xyz, points, W0, b0, gamma0, beta0, W1, b1, gamma1, beta1, W2, b2, gamma2, beta2):
    new_xyz = _fps_new_xyz(xyz)
    idx = _ball_query(xyz, new_xyz)
    P, Q = _prep(xyz, points, new_xyz, W0, b0)
    flat_idx = (idx + (jnp.arange(_B, dtype=jnp.int32) * _N)[:, None, None]
                ).reshape(-1)
    g = jnp.take(P.reshape(_B * _N, 32), flat_idx, axis=0)
    z1 = g - jnp.repeat(Q.reshape(_B * _NPOINT, 32), _NSAMPLE, axis=0)
    stats1 = jnp.stack([jnp.sum(z1, axis=0), jnp.sum(z1 * z1, axis=0)])
    z2, stats2 = _mlp_mid(z1, stats1, W1, b1, gamma0, beta0)
    m, stats3 = _mlp_last(z2, stats2, W2, b2, gamma1, beta1)
    out = _final(m, stats3, gamma2, beta2)
    return new_xyz, out.reshape(_B, _NPOINT, 64)


# speedup vs baseline: 14.6174x; 1.6265x over previous
import functools

import jax
import jax.numpy as jnp
from jax.experimental import pallas as pl
from jax.experimental.pallas import tpu as pltpu

_NPOINT = 1024
_RADIUS = 0.1
_NSAMPLE = 32
_BN_EPS = 1e-5

_B = 4
_N = 8192
_ROWS = 64
_COLS = 128


def _fps_body(xyz_ref, xyzs_ref, nx_ref, ny_ref, nz_ref):
    iota = (jax.lax.broadcasted_iota(jnp.int32, (_ROWS, _COLS), 0) * _COLS
            + jax.lax.broadcasted_iota(jnp.int32, (_ROWS, _COLS), 1))

    def body(i, state):
        dists, farthest = state
        cs = []
        for b in range(_B):
            cx = xyzs_ref[b, 0, farthest[b]]
            cy = xyzs_ref[b, 1, farthest[b]]
            cz = xyzs_ref[b, 2, farthest[b]]
            nx_ref[b, i] = cx
            ny_ref[b, i] = cy
            nz_ref[b, i] = cz
            cs.append((cx, cy, cz))
        new_d = []
        for b in range(_B):
            dx = xyz_ref[b, 0] - cs[b][0]
            dy = xyz_ref[b, 1] - cs[b][1]
            dz = xyz_ref[b, 2] - cs[b][2]
            d = dx * dx + dy * dy
            d = d + dz * dz
            new_d.append(jnp.minimum(dists[b], d))
        ms = [jnp.max(new_d[b]) for b in range(_B)]
        new_f = [jnp.min(jnp.where(new_d[b] == ms[b], iota, _N))
                 for b in range(_B)]
        return tuple(new_d), tuple(new_f)

    dists0 = tuple(jnp.full((_ROWS, _COLS), 1e10, dtype=jnp.float32)
                   for _ in range(_B))
    far0 = tuple(jnp.int32(0) for _ in range(_B))
    jax.lax.fori_loop(0, _NPOINT, body, (dists0, far0))


def _fps_new_xyz(xyz):
    xt = xyz.transpose(0, 2, 1)
    xtv = xt.reshape(_B, 3, _ROWS, _COLS)
    out_sds = jax.ShapeDtypeStruct((_B, _NPOINT), jnp.float32)
    smem_spec = pl.BlockSpec(memory_space=pltpu.SMEM)
    nx, ny, nz = pl.pallas_call(
        _fps_body,
        in_specs=[pl.BlockSpec(memory_space=pltpu.VMEM), smem_spec],
        out_shape=(out_sds, out_sds, out_sds),
        out_specs=(smem_spec, smem_spec, smem_spec),
    )(xtv, xt)
    return jnp.stack([nx, ny, nz], axis=-1)


_QT = 128


def _bq_body(q_ref, p_ref, out_ref):
    q = q_ref[0]
    qx, qy, qz = q[:, 0:1], q[:, 1:2], q[:, 2:3]
    px = p_ref[0, 0:1, :]
    py = p_ref[0, 1:2, :]
    pz = p_ref[0, 2:3, :]
    qq = (qx * qx + qy * qy) + qz * qz
    pp = (px * px + py * py) + pz * pz
    qp = jax.lax.dot_general(q, p_ref[0], (((1,), (0,)), ((), ())),
                             preferred_element_type=jnp.float32)
    d2 = (qq + pp) - 2.0 * qp
    mask = jnp.where(d2 < _RADIUS * _RADIUS, 1.0, 0.0)

    ch = 128
    nch = _N // ch
    li = jax.lax.broadcasted_iota(jnp.int32, (ch, ch), 0)
    lj = jax.lax.broadcasted_iota(jnp.int32, (ch, ch), 1)
    ltri = jnp.where(li <= lj, 1.0, 0.0)
    base = jnp.zeros((_QT, 1), jnp.float32)
    psums = []
    for c in range(nch):
        mc = mask[:, c * ch:(c + 1) * ch]
        lsum = jax.lax.dot(mc, ltri, precision=jax.lax.Precision.HIGHEST)
        psums.append(lsum + base)
        base = base + lsum[:, ch - 1:ch]
    psum = jnp.concatenate(psums, axis=-1)
    cnt = base

    u = jnp.minimum(psum, 33.0)
    cols = [jnp.sum(jnp.where(u <= float(k), 1.0, 0.0), axis=-1)
            for k in range(_NSAMPLE)]
    idx = jnp.stack(cols, axis=-1)
    first = jnp.where(cnt > 0.0, idx[:, 0:1], 0.0)
    krange = jax.lax.broadcasted_iota(jnp.int32, (_QT, _NSAMPLE), 1)
    out_ref[0] = jnp.where(krange < cnt.astype(jnp.int32), idx, first).astype(jnp.int32)


def _ball_query(xyz, new_xyz):
    xt = xyz.transpose(0, 2, 1)
    grid = (_B, _NPOINT // _QT)
    return pl.pallas_call(
        _bq_body,
        grid=grid,
        in_specs=[
            pl.BlockSpec((1, _QT, 3), lambda b, s: (b, s, 0)),
            pl.BlockSpec((1, 3, _N), lambda b, s: (b, 0, 0)),
        ],
        out_specs=pl.BlockSpec((1, _QT, _NSAMPLE), lambda b, s: (b, s, 0)),
        out_shape=jax.ShapeDtypeStruct((_B, _NPOINT, _NSAMPLE), jnp.int32),
    )(new_xyz, xt)


def _prep_body(cat_ref, q_ref, w0_ref, b0_ref, p_out, q_out):
    p_out[0] = jax.lax.dot_general(
        cat_ref[0], w0_ref[...], (((1,), (0,)), ((), ())),
        preferred_element_type=jnp.float32) + b0_ref[...]
    q_out[0] = jax.lax.dot_general(
        q_ref[0], w0_ref[0:3, :], (((1,), (0,)), ((), ())),
        preferred_element_type=jnp.float32)


def _prep(xyz, points, new_xyz, W0, b0):
    cat = jnp.concatenate([xyz, points], axis=-1)
    return pl.pallas_call(
        _prep_body,
        grid=(_B,),
        in_specs=[
            pl.BlockSpec((1, _N, 19), lambda b: (b, 0, 0)),
            pl.BlockSpec((1, _NPOINT, 3), lambda b: (b, 0, 0)),
            pl.BlockSpec((19, 32), lambda b: (0, 0)),
            pl.BlockSpec((32,), lambda b: (0,)),
        ],
        out_specs=(
            pl.BlockSpec((1, _N, 32), lambda b: (b, 0, 0)),
            pl.BlockSpec((1, _NPOINT, 32), lambda b: (b, 0, 0)),
        ),
        out_shape=(
            jax.ShapeDtypeStruct((_B, _N, 32), jnp.float32),
            jax.ShapeDtypeStruct((_B, _NPOINT, 32), jnp.float32),
        ),
    )(cat, new_xyz, W0, b0)


_RT = 2048
_R = _B * _NPOINT * _NSAMPLE


def _bn_stats(s_ref, cin):
    n = float(_R)
    mean = s_ref[0:1, :] / n
    var = s_ref[1:2, :] / n - mean * mean
    inv = jax.lax.rsqrt(var + _BN_EPS)
    return mean, inv


def _mlp_mid_body(z_ref, s_ref, w_ref, b_ref, g_ref, be_ref, out_ref,
                  sout_ref, acc_ref):
    t = pl.program_id(0)

    @pl.when(t == 0)
    def _():
        acc_ref[...] = jnp.zeros_like(acc_ref)

    z = z_ref[...]
    mean, inv = _bn_stats(s_ref, z.shape[-1])
    a = jnp.maximum((z - mean) * inv * g_ref[...] + be_ref[...], 0.0)
    z2 = jax.lax.dot_general(a, w_ref[...], (((1,), (0,)), ((), ())),
                             preferred_element_type=jnp.float32) + b_ref[...]
    out_ref[...] = z2
    acc_ref[0:1, :] += jnp.sum(z2, axis=0, keepdims=True)
    acc_ref[1:2, :] += jnp.sum(z2 * z2, axis=0, keepdims=True)

    @pl.when(t == pl.num_programs(0) - 1)
    def _():
        sout_ref[...] = acc_ref[...]


def _mlp_mid(z1, stats1, W1, b1, g1, be1):
    cout = W1.shape[1]
    grid = (_R // _RT,)
    return pl.pallas_call(
        _mlp_mid_body,
        grid=grid,
        in_specs=[
            pl.BlockSpec((_RT, 32), lambda t: (t, 0)),
            pl.BlockSpec((2, 32), lambda t: (0, 0)),
            pl.BlockSpec((32, cout), lambda t: (0, 0)),
            pl.BlockSpec((cout,), lambda t: (0,)),
            pl.BlockSpec((cout,), lambda t: (0,)),
            pl.BlockSpec((cout,), lambda t: (0,)),
        ],
        out_specs=(
            pl.BlockSpec((_RT, cout), lambda t: (t, 0)),
            pl.BlockSpec((2, cout), lambda t: (0, 0)),
        ),
        out_shape=(
            jax.ShapeDtypeStruct((_R, cout), jnp.float32),
            jax.ShapeDtypeStruct((2, cout), jnp.float32),
        ),
        scratch_shapes=[pltpu.VMEM((2, cout), jnp.float32)],
    )(z1, stats1, W1, b1, g1, be1)


_QT3 = 64


def _mlp_last_body(z_ref, s_ref, w_ref, b_ref, g_ref, be_ref, out_ref,
                   sout_ref, acc_ref):
    t = pl.program_id(0)

    @pl.when(t == 0)
    def _():
        acc_ref[...] = jnp.zeros_like(acc_ref)

    z = z_ref[...].reshape(_QT3 * _NSAMPLE, 32)
    mean, inv = _bn_stats(s_ref, 32)
    a = jnp.maximum((z - mean) * inv * g_ref[...] + be_ref[...], 0.0)
    z3 = jax.lax.dot_general(a, w_ref[...], (((1,), (0,)), ((), ())),
                             preferred_element_type=jnp.float32) + b_ref[...]
    acc_ref[0:1, :] += jnp.sum(z3, axis=0, keepdims=True)
    acc_ref[1:2, :] += jnp.sum(z3 * z3, axis=0, keepdims=True)
    out_ref[...] = jnp.max(z3.reshape(_QT3, _NSAMPLE, 64), axis=1)

    @pl.when(t == pl.num_programs(0) - 1)
    def _():
        sout_ref[...] = acc_ref[...]


def _mlp_last(z2, stats2, W2, b2, g2, be2):
    grid = (_B * _NPOINT // _QT3,)
    z2r = z2.reshape(_B * _NPOINT, _NSAMPLE, 32)
    return pl.pallas_call(
        _mlp_last_body,
        grid=grid,
        in_specs=[
            pl.BlockSpec((_QT3, _NSAMPLE, 32), lambda t: (t, 0, 0)),
            pl.BlockSpec((2, 32), lambda t: (0, 0)),
            pl.BlockSpec((32, 64), lambda t: (0, 0)),
            pl.BlockSpec((64,), lambda t: (0,)),
            pl.BlockSpec((32,), lambda t: (0,)),
            pl.BlockSpec((32,), lambda t: (0,)),
        ],
        out_specs=(
            pl.BlockSpec((_QT3, 64), lambda t: (t, 0)),
            pl.BlockSpec((2, 64), lambda t: (0, 0)),
        ),
        out_shape=(
            jax.ShapeDtypeStruct((_B * _NPOINT, 64), jnp.float32),
            jax.ShapeDtypeStruct((2, 64), jnp.float32),
        ),
        scratch_shapes=[pltpu.VMEM((2, 64), jnp.float32)],
    )(z2r, stats2, W2, b2, g2, be2)


def _final_body(m_ref, s_ref, g_ref, be_ref, out_ref):
    mean, inv = _bn_stats(s_ref, 64)
    out_ref[...] = jnp.maximum(
        (m_ref[...] - mean) * inv * g_ref[...] + be_ref[...], 0.0)


def _final(m, stats3, g2, be2):
    return pl.pallas_call(
        _final_body,
        in_specs=[
            pl.BlockSpec((_B * _NPOINT, 64), lambda: (0, 0)),
            pl.BlockSpec((2, 64), lambda: (0, 0)),
            pl.BlockSpec((64,), lambda: (0,)),
            pl.BlockSpec((64,), lambda: (0,)),
        ],
        out_specs=pl.BlockSpec((_B * _NPOINT, 64), lambda: (0, 0)),
        out_shape=jax.ShapeDtypeStruct((_B * _NPOINT, 64), jnp.float32),
    )(m, stats3, g2, be2)


def kernel(xyz, points, W0, b0, gamma0, beta0, W1, b1, gamma1, beta1, W2, b2, gamma2, beta2):
    new_xyz = _fps_new_xyz(xyz)
    idx = _ball_query(xyz, new_xyz)
    P, Q = _prep(xyz, points, new_xyz, W0, b0)
    flat_idx = (idx + (jnp.arange(_B, dtype=jnp.int32) * _N)[:, None, None]
                ).reshape(-1)
    g = jnp.take(P.reshape(_B * _N, 32), flat_idx, axis=0)
    z1 = g - jnp.repeat(Q.reshape(_B * _NPOINT, 32), _NSAMPLE, axis=0)
    stats1 = jnp.stack([jnp.sum(z1, axis=0), jnp.sum(z1 * z1, axis=0)])
    z2, stats2 = _mlp_mid(z1, stats1, W1, b1, gamma0, beta0)
    m, stats3 = _mlp_last(z2, stats2, W2, b2, gamma1, beta1)
    out = _final(m, stats3, gamma2, beta2)
    return new_xyz, out.reshape(_B, _NPOINT, 64)


# speedup vs baseline: 19.1805x; 1.3122x over previous
import functools

import jax
import jax.numpy as jnp
from jax import lax
from jax.experimental import pallas as pl
from jax.experimental.pallas import tpu as pltpu
from jax.experimental.pallas import tpu_sc as plsc

_NPOINT = 1024
_RADIUS = 0.1
_NSAMPLE = 32
_BN_EPS = 1e-5

_B = 4
_N = 8192
_ROWS = 64
_COLS = 128


def _fps_body(xyz_ref, xyzs_ref, nx_ref, ny_ref, nz_ref):
    iota = (jax.lax.broadcasted_iota(jnp.int32, (_ROWS, _COLS), 0) * _COLS
            + jax.lax.broadcasted_iota(jnp.int32, (_ROWS, _COLS), 1))

    def body(i, state):
        dists, farthest = state
        cs = []
        for b in range(_B):
            cx = xyzs_ref[b, 0, farthest[b]]
            cy = xyzs_ref[b, 1, farthest[b]]
            cz = xyzs_ref[b, 2, farthest[b]]
            nx_ref[b, i] = cx
            ny_ref[b, i] = cy
            nz_ref[b, i] = cz
            cs.append((cx, cy, cz))
        new_d = []
        for b in range(_B):
            dx = xyz_ref[b, 0] - cs[b][0]
            dy = xyz_ref[b, 1] - cs[b][1]
            dz = xyz_ref[b, 2] - cs[b][2]
            d = dx * dx + dy * dy
            d = d + dz * dz
            new_d.append(jnp.minimum(dists[b], d))
        ms = [jnp.max(new_d[b]) for b in range(_B)]
        new_f = [jnp.min(jnp.where(new_d[b] == ms[b], iota, _N))
                 for b in range(_B)]
        return tuple(new_d), tuple(new_f)

    dists0 = tuple(jnp.full((_ROWS, _COLS), 1e10, dtype=jnp.float32)
                   for _ in range(_B))
    far0 = tuple(jnp.int32(0) for _ in range(_B))
    jax.lax.fori_loop(0, _NPOINT, body, (dists0, far0))


def _fps_new_xyz(xyz):
    xt = xyz.transpose(0, 2, 1)
    xtv = xt.reshape(_B, 3, _ROWS, _COLS)
    out_sds = jax.ShapeDtypeStruct((_B, _NPOINT), jnp.float32)
    smem_spec = pl.BlockSpec(memory_space=pltpu.SMEM)
    nx, ny, nz = pl.pallas_call(
        _fps_body,
        in_specs=[pl.BlockSpec(memory_space=pltpu.VMEM), smem_spec],
        out_shape=(out_sds, out_sds, out_sds),
        out_specs=(smem_spec, smem_spec, smem_spec),
    )(xtv, xt)
    return jnp.stack([nx, ny, nz], axis=-1)


_QT = 128


def _bq_body(q_ref, p_ref, out_ref):
    q = q_ref[0]
    qx, qy, qz = q[:, 0:1], q[:, 1:2], q[:, 2:3]
    px = p_ref[0, 0:1, :]
    py = p_ref[0, 1:2, :]
    pz = p_ref[0, 2:3, :]
    qq = (qx * qx + qy * qy) + qz * qz
    pp = (px * px + py * py) + pz * pz
    qp = jax.lax.dot_general(q, p_ref[0], (((1,), (0,)), ((), ())),
                             preferred_element_type=jnp.float32)
    d2 = (qq + pp) - 2.0 * qp
    mask = jnp.where(d2 < _RADIUS * _RADIUS, 1.0, 0.0)

    ch = 128
    nch = _N // ch
    li = jax.lax.broadcasted_iota(jnp.int32, (ch, ch), 0)
    lj = jax.lax.broadcasted_iota(jnp.int32, (ch, ch), 1)
    ltri = jnp.where(li <= lj, 1.0, 0.0)
    base = jnp.zeros((_QT, 1), jnp.float32)
    psums = []
    for c in range(nch):
        mc = mask[:, c * ch:(c + 1) * ch]
        lsum = jax.lax.dot(mc, ltri, precision=jax.lax.Precision.HIGHEST)
        psums.append(lsum + base)
        base = base + lsum[:, ch - 1:ch]
    psum = jnp.concatenate(psums, axis=-1)
    cnt = base

    u = jnp.minimum(psum, 33.0)
    cols = [jnp.sum(jnp.where(u <= float(k), 1.0, 0.0), axis=-1)
            for k in range(_NSAMPLE)]
    idx = jnp.stack(cols, axis=-1)
    first = jnp.where(cnt > 0.0, idx[:, 0:1], 0.0)
    krange = jax.lax.broadcasted_iota(jnp.int32, (_QT, _NSAMPLE), 1)
    out_ref[0] = jnp.where(krange < cnt.astype(jnp.int32), idx, first).astype(jnp.int32)


def _ball_query(xyz, new_xyz):
    xt = xyz.transpose(0, 2, 1)
    grid = (_B, _NPOINT // _QT)
    return pl.pallas_call(
        _bq_body,
        grid=grid,
        in_specs=[
            pl.BlockSpec((1, _QT, 3), lambda b, s: (b, s, 0)),
            pl.BlockSpec((1, 3, _N), lambda b, s: (b, 0, 0)),
        ],
        out_specs=pl.BlockSpec((1, _QT, _NSAMPLE), lambda b, s: (b, s, 0)),
        out_shape=jax.ShapeDtypeStruct((_B, _NPOINT, _NSAMPLE), jnp.int32),
    )(new_xyz, xt)


def _prep_body(cat_ref, q_ref, w0_ref, b0_ref, p_out, q_out):
    p_out[0] = jax.lax.dot_general(
        cat_ref[0], w0_ref[...], (((1,), (0,)), ((), ())),
        preferred_element_type=jnp.float32) + b0_ref[...]
    q_out[0] = jax.lax.dot_general(
        q_ref[0], w0_ref[0:3, :], (((1,), (0,)), ((), ())),
        preferred_element_type=jnp.float32)


def _prep(xyz, points, new_xyz, W0, b0):
    cat = jnp.concatenate([xyz, points], axis=-1)
    return pl.pallas_call(
        _prep_body,
        grid=(_B,),
        in_specs=[
            pl.BlockSpec((1, _N, 19), lambda b: (b, 0, 0)),
            pl.BlockSpec((1, _NPOINT, 3), lambda b: (b, 0, 0)),
            pl.BlockSpec((19, 32), lambda b: (0, 0)),
            pl.BlockSpec((32,), lambda b: (0,)),
        ],
        out_specs=(
            pl.BlockSpec((1, _N, 32), lambda b: (b, 0, 0)),
            pl.BlockSpec((1, _NPOINT, 32), lambda b: (b, 0, 0)),
        ),
        out_shape=(
            jax.ShapeDtypeStruct((_B, _N, 32), jnp.float32),
            jax.ShapeDtypeStruct((_B, _NPOINT, 32), jnp.float32),
        ),
    )(cat, new_xyz, W0, b0)


_RT = 2048
_R = _B * _NPOINT * _NSAMPLE

_NW = 32
_CH = 128


def _sc_gather_body(tbl_ref, idx_ref, q_ref, z_out, st_out,
                    idx_v, q_v, rows_v, sacc, sem):
    wid = lax.axis_index("s") * 2 + lax.axis_index("c")
    nchunks = _R // (_NW * _CH)
    qpc = _CH // _NSAMPLE

    pltpu.sync_copy(idx_ref.at[pl.ds(wid * nchunks, nchunks)], idx_v)
    pltpu.sync_copy(q_ref.at[pl.ds(wid * nchunks * qpc, nchunks * qpc)], q_v)

    zvec = jnp.zeros((16,), jnp.float32)

    def chunk(j, acc):
        s0, s1, ss0, ss1 = acc
        pltpu.async_copy(tbl_ref.at[idx_v.at[j]], rows_v, sem).wait()
        for r in range(_CH):
            qrow = r // _NSAMPLE
            z0 = rows_v[r, 0:16] - q_v[j * qpc + qrow, 0:16]
            z1 = rows_v[r, 16:32] - q_v[j * qpc + qrow, 16:32]
            rows_v[r, 0:16] = z0
            rows_v[r, 16:32] = z1
            s0 = s0 + z0
            s1 = s1 + z1
            ss0 = ss0 + z0 * z0
            ss1 = ss1 + z1 * z1
        pltpu.sync_copy(rows_v, z_out.at[pl.ds((wid * nchunks + j) * _CH, _CH)])
        return (s0, s1, ss0, ss1)

    s0, s1, ss0, ss1 = lax.fori_loop(0, nchunks, chunk,
                                     (zvec, zvec, zvec, zvec))
    sacc[0, 0:16] = s0
    sacc[0, 16:32] = s1
    sacc[1, 0:16] = ss0
    sacc[1, 16:32] = ss1
    pltpu.sync_copy(sacc, st_out.at[wid])


def _sc_gather_z1(P2, flat_idx, Q2):
    import functools
    mesh = plsc.VectorSubcoreMesh(core_axis_name="c", subcore_axis_name="s")
    idx2 = flat_idx.reshape(_R // _CH, _CH)
    nchunks = _R // (_NW * _CH)
    qpc = _CH // _NSAMPLE
    k = functools.partial(
        pl.kernel, mesh=mesh,
        compiler_params=pltpu.CompilerParams(use_tc_tiling_on_sc=False),
        out_type=(
            jax.ShapeDtypeStruct((_R, 32), jnp.float32),
            jax.ShapeDtypeStruct((_NW, 2, 32), jnp.float32),
        ),
        scratch_types=[
            pltpu.VMEM((nchunks, _CH), jnp.int32),
            pltpu.VMEM((nchunks * qpc, 32), jnp.float32),
            pltpu.VMEM((_CH, 32), jnp.float32),
            pltpu.VMEM((2, 32), jnp.float32),
            pltpu.SemaphoreType.DMA,
        ],
    )(_sc_gather_body)
    z1, st = k(P2, idx2, Q2)
    stats1 = jnp.sum(st, axis=0)
    return z1, stats1


def _bn_stats(s_ref, cin):
    n = float(_R)
    mean = s_ref[0:1, :] / n
    var = s_ref[1:2, :] / n - mean * mean
    inv = jax.lax.rsqrt(var + _BN_EPS)
    return mean, inv


def _mlp_mid_body(z_ref, s_ref, w_ref, b_ref, g_ref, be_ref, out_ref,
                  sout_ref, acc_ref):
    t = pl.program_id(0)

    @pl.when(t == 0)
    def _():
        acc_ref[...] = jnp.zeros_like(acc_ref)

    z = z_ref[...]
    mean, inv = _bn_stats(s_ref, z.shape[-1])
    a = jnp.maximum((z - mean) * inv * g_ref[...] + be_ref[...], 0.0)
    z2 = jax.lax.dot_general(a, w_ref[...], (((1,), (0,)), ((), ())),
                             preferred_element_type=jnp.float32) + b_ref[...]
    out_ref[...] = z2
    acc_ref[0:1, :] += jnp.sum(z2, axis=0, keepdims=True)
    acc_ref[1:2, :] += jnp.sum(z2 * z2, axis=0, keepdims=True)

    @pl.when(t == pl.num_programs(0) - 1)
    def _():
        sout_ref[...] = acc_ref[...]


def _mlp_mid(z1, stats1, W1, b1, g1, be1):
    cout = W1.shape[1]
    grid = (_R // _RT,)
    return pl.pallas_call(
        _mlp_mid_body,
        grid=grid,
        in_specs=[
            pl.BlockSpec((_RT, 32), lambda t: (t, 0)),
            pl.BlockSpec((2, 32), lambda t: (0, 0)),
            pl.BlockSpec((32, cout), lambda t: (0, 0)),
            pl.BlockSpec((cout,), lambda t: (0,)),
            pl.BlockSpec((cout,), lambda t: (0,)),
            pl.BlockSpec((cout,), lambda t: (0,)),
        ],
        out_specs=(
            pl.BlockSpec((_RT, cout), lambda t: (t, 0)),
            pl.BlockSpec((2, cout), lambda t: (0, 0)),
        ),
        out_shape=(
            jax.ShapeDtypeStruct((_R, cout), jnp.float32),
            jax.ShapeDtypeStruct((2, cout), jnp.float32),
        ),
        scratch_shapes=[pltpu.VMEM((2, cout), jnp.float32)],
    )(z1, stats1, W1, b1, g1, be1)


_QT3 = 64


def _mlp_last_body(z_ref, s_ref, w_ref, b_ref, g_ref, be_ref, out_ref,
                   sout_ref, acc_ref):
    t = pl.program_id(0)

    @pl.when(t == 0)
    def _():
        acc_ref[...] = jnp.zeros_like(acc_ref)

    z = z_ref[...].reshape(_QT3 * _NSAMPLE, 32)
    mean, inv = _bn_stats(s_ref, 32)
    a = jnp.maximum((z - mean) * inv * g_ref[...] + be_ref[...], 0.0)
    z3 = jax.lax.dot_general(a, w_ref[...], (((1,), (0,)), ((), ())),
                             preferred_element_type=jnp.float32) + b_ref[...]
    acc_ref[0:1, :] += jnp.sum(z3, axis=0, keepdims=True)
    acc_ref[1:2, :] += jnp.sum(z3 * z3, axis=0, keepdims=True)
    out_ref[...] = jnp.max(z3.reshape(_QT3, _NSAMPLE, 64), axis=1)

    @pl.when(t == pl.num_programs(0) - 1)
    def _():
        sout_ref[...] = acc_ref[...]


def _mlp_last(z2, stats2, W2, b2, g2, be2):
    grid = (_B * _NPOINT // _QT3,)
    z2r = z2.reshape(_B * _NPOINT, _NSAMPLE, 32)
    return pl.pallas_call(
        _mlp_last_body,
        grid=grid,
        in_specs=[
            pl.BlockSpec((_QT3, _NSAMPLE, 32), lambda t: (t, 0, 0)),
            pl.BlockSpec((2, 32), lambda t: (0, 0)),
            pl.BlockSpec((32, 64), lambda t: (0, 0)),
            pl.BlockSpec((64,), lambda t: (0,)),
            pl.BlockSpec((32,), lambda t: (0,)),
            pl.BlockSpec((32,), lambda t: (0,)),
        ],
        out_specs=(
            pl.BlockSpec((_QT3, 64), lambda t: (t, 0)),
            pl.BlockSpec((2, 64), lambda t: (0, 0)),
        ),
        out_shape=(
            jax.ShapeDtypeStruct((_B * _NPOINT, 64), jnp.float32),
            jax.ShapeDtypeStruct((2, 64), jnp.float32),
        ),
        scratch_shapes=[pltpu.VMEM((2, 64), jnp.float32)],
    )(z2r, stats2, W2, b2, g2, be2)


def _final_body(m_ref, s_ref, g_ref, be_ref, out_ref):
    mean, inv = _bn_stats(s_ref, 64)
    out_ref[...] = jnp.maximum(
        (m_ref[...] - mean) * inv * g_ref[...] + be_ref[...], 0.0)


def _final(m, stats3, g2, be2):
    return pl.pallas_call(
        _final_body,
        in_specs=[
            pl.BlockSpec((_B * _NPOINT, 64), lambda: (0, 0)),
            pl.BlockSpec((2, 64), lambda: (0, 0)),
            pl.BlockSpec((64,), lambda: (0,)),
            pl.BlockSpec((64,), lambda: (0,)),
        ],
        out_specs=pl.BlockSpec((_B * _NPOINT, 64), lambda: (0, 0)),
        out_shape=jax.ShapeDtypeStruct((_B * _NPOINT, 64), jnp.float32),
    )(m, stats3, g2, be2)


def kernel(xyz, points, W0, b0, gamma0, beta0, W1, b1, gamma1, beta1, W2, b2, gamma2, beta2):
    new_xyz = _fps_new_xyz(xyz)
    idx = _ball_query(xyz, new_xyz)
    P, Q = _prep(xyz, points, new_xyz, W0, b0)
    flat_idx = (idx + (jnp.arange(_B, dtype=jnp.int32) * _N)[:, None, None]
                ).reshape(-1)
    z1, stats1 = _sc_gather_z1(P.reshape(_B * _N, 32), flat_idx,
                               Q.reshape(_B * _NPOINT, 32))
    z2, stats2 = _mlp_mid(z1, stats1, W1, b1, gamma0, beta0)
    m, stats3 = _mlp_last(z2, stats2, W2, b2, gamma1, beta1)
    out = _final(m, stats3, gamma2, beta2)
    return new_xyz, out.reshape(_B, _NPOINT, 64)
